# baseline (speedup 1.0000x reference)
import functools

import jax
import jax.numpy as jnp
from jax import lax
from jax.experimental import pallas as pl
from jax.experimental.pallas import tpu as pltpu
from jax.experimental.pallas import tpu_sc as plsc

B = 4096
L = 200
VOCAB = 1000
VPAD = 1024
DIM = 128

NC = 2
NS = 16
LANES = 16
NW = NC * NS

NCHUNK = 1
NB = B // NCHUNK

HIST_W = LANES * VPAD


def _sc_hist_kernel(rows_w, xg_hbm, counts_hbm, x_v, hist_v, sem0, sem1):
    groups = rows_w // LANES
    wid = lax.axis_index("s") * NC + lax.axis_index("c")
    base = wid * rows_w
    pltpu.sync_copy(xg_hbm.at[wid], x_v)

    lane_off = lax.iota(jnp.int32, LANES) * VPAD
    ones = jnp.full((LANES,), 1.0, dtype=jnp.float32)
    zeros = jnp.zeros((LANES,), dtype=jnp.float32)
    sems = [sem0, sem1]
    pending = [None, None]

    for g in range(groups):
        b = g % 2
        boff = b * HIST_W
        if pending[b] is not None:
            pending[b].wait()
        lane_off_g = lane_off + boff

        if g < 2:
            def _zero(i, _, boff=boff):
                for u in range(16):
                    hist_v[pl.ds(boff + (i * 16 + u) * LANES, LANES)] = zeros
                return 0

            lax.fori_loop(0, HIST_W // LANES // 16, _zero, 0)
        else:
            def _unzero(t, _, go=g - 2, lane_off_g=lane_off_g):
                for u in range(8):
                    toks = x_v[pl.ds((t * 8 + u) * rows_w + go * LANES, LANES)]
                    plsc.store_scatter(hist_v, [lane_off_g + toks], zeros)
                return 0

            lax.fori_loop(0, L // 8, _unzero, 0)

        def _scat(t, _, g=g, lane_off_g=lane_off_g):
            for u in range(8):
                toks = x_v[pl.ds((t * 8 + u) * rows_w + g * LANES, LANES)]
                plsc.addupdate_scatter(hist_v, [lane_off_g + toks], ones)
            return 0

        lax.fori_loop(0, L // 8, _scat, 0)

        pending[b] = pltpu.async_copy(
            hist_v.at[pl.ds(boff, HIST_W)],
            counts_hbm.at[pl.ds((base + g * LANES) * VPAD, HIST_W)],
            sems[b])

    pending[0].wait()
    if pending[1] is not None:
        pending[1].wait()


def _sc_hist(xg):
    rows_w = xg.shape[1] // L
    mesh = plsc.VectorSubcoreMesh(core_axis_name="c", subcore_axis_name="s")
    return pl.kernel(
        functools.partial(_sc_hist_kernel, rows_w),
        out_type=jax.ShapeDtypeStruct((NW * rows_w * VPAD,), jnp.float32),
        mesh=mesh,
        scratch_types=[
            pltpu.VMEM((L * rows_w,), jnp.int32),
            pltpu.VMEM((2 * HIST_W,), jnp.float32),
            pltpu.SemaphoreType.DMA,
            pltpu.SemaphoreType.DMA,
        ],
        compiler_params=pltpu.CompilerParams(needs_layout_passes=False),
    )(xg)


def _tc_mlp_kernel(counts_ref, emb_ref, W1_ref, b1_ref, W2_ref, b2_ref,
                   W3_ref, b3_ref, out_ref):
    m = jnp.dot(counts_ref[...], emb_ref[...],
                preferred_element_type=jnp.float32) * (1.0 / L)
    h = jnp.maximum(jnp.dot(m, W1_ref[...],
                            preferred_element_type=jnp.float32) + b1_ref[...],
                    0.0)
    h2 = jnp.dot(h, W2_ref[...], preferred_element_type=jnp.float32) + b2_ref[...]
    h2 = jnp.where(h2 >= 0, h2, 0.01 * h2)
    out_ref[...] = jnp.dot(h2, W3_ref[...],
                           preferred_element_type=jnp.float32) + b3_ref[...]


def _tc_mlp(counts, emb_pad, W1, b1, W2, b2, W3, b3):
    nb = counts.shape[0]
    BM = 512
    grid = (nb // BM,)
    return pl.pallas_call(
        _tc_mlp_kernel,
        grid=grid,
        in_specs=[
            pl.BlockSpec((BM, VPAD), lambda i: (i, 0)),
            pl.BlockSpec((VPAD, DIM), lambda i: (0, 0)),
            pl.BlockSpec(W1.shape, lambda i: (0, 0)),
            pl.BlockSpec(b1.shape, lambda i: (0, 0)),
            pl.BlockSpec(W2.shape, lambda i: (0, 0)),
            pl.BlockSpec(b2.shape, lambda i: (0, 0)),
            pl.BlockSpec(W3.shape, lambda i: (0, 0)),
            pl.BlockSpec(b3.shape, lambda i: (0, 0)),
        ],
        out_specs=pl.BlockSpec((BM, 2), lambda i: (i, 0)),
        out_shape=jax.ShapeDtypeStruct((nb, 2), jnp.float32),
    )(counts, emb_pad, W1, b1, W2, b2, W3, b3)


def kernel(x, length, emb, W1, b1, W2, b2, W3, b3):
    del length
    emb_pad = jnp.pad(emb, ((0, VPAD - VOCAB), (0, 0)))
    b1r = b1.reshape(1, -1)
    b2r = b2.reshape(1, -1)
    b3r = b3.reshape(1, -1)

    rows_w = NB // NW
    outs = []
    for c in range(NCHUNK):
        xc = lax.slice_in_dim(x.astype(jnp.int32), c * NB, (c + 1) * NB, axis=0)
        xg = xc.reshape(NW, rows_w, L).transpose(0, 2, 1).reshape(NW, L * rows_w)
        counts = _sc_hist(xg).reshape(NB, VPAD)
        outs.append(_tc_mlp(counts, emb_pad, W1, b1r, W2, b2r, W3, b3r))
    return jnp.concatenate(outs, axis=0)


# speedup vs baseline: 1.0566x; 1.0566x over previous
import functools

import jax
import jax.numpy as jnp
from jax import lax
from jax.experimental import pallas as pl
from jax.experimental.pallas import tpu as pltpu
from jax.experimental.pallas import tpu_sc as plsc

B = 4096
L = 200
VOCAB = 1000
VPAD = 1024
DIM = 128

NC = 2
NS = 16
LANES = 16
NW = NC * NS

NCHUNK = 1
NB = B // NCHUNK

HIST_W = LANES * VPAD


def _sc_hist_kernel(rows_w, xg_hbm, counts_hbm, x_v, hist_v,
                    sem0, sem1, sem2, sem3):
    groups = rows_w // LANES
    wid = lax.axis_index("s") * NC + lax.axis_index("c")
    base = wid * rows_w
    pltpu.sync_copy(xg_hbm.at[wid], x_v)

    lane_off = lax.iota(jnp.int32, LANES) * VPAD
    ones = jnp.full((LANES,), 1.0, dtype=jnp.float32)
    zeros = jnp.zeros((LANES,), dtype=jnp.float32)
    sems = [sem0, sem1, sem2, sem3]
    pending = [None] * len(sems)

    nbuf = len(sems)
    for g in range(groups):
        b = g % nbuf
        boff = b * HIST_W
        if pending[b] is not None:
            pending[b].wait()
        lane_off_g = lane_off + boff

        def _zero(i, _, boff=boff):
            for u in range(16):
                hist_v[pl.ds(boff + (i * 16 + u) * LANES, LANES)] = zeros
            return 0

        lax.fori_loop(0, HIST_W // LANES // 16, _zero, 0)

        def _scat(t, _, g=g, lane_off_g=lane_off_g):
            for u in range(8):
                toks = x_v[pl.ds((t * 8 + u) * rows_w + g * LANES, LANES)]
                plsc.addupdate_scatter(hist_v, [lane_off_g + toks], ones)
            return 0

        lax.fori_loop(0, L // 8, _scat, 0)

        pending[b] = pltpu.async_copy(
            hist_v.at[pl.ds(boff, HIST_W)],
            counts_hbm.at[pl.ds((base + g * LANES) * VPAD, HIST_W)],
            sems[b])

    for p in pending:
        if p is not None:
            p.wait()


def _sc_hist(xg):
    rows_w = xg.shape[1] // L
    mesh = plsc.VectorSubcoreMesh(core_axis_name="c", subcore_axis_name="s")
    return pl.kernel(
        functools.partial(_sc_hist_kernel, rows_w),
        out_type=jax.ShapeDtypeStruct((NW * rows_w * VPAD,), jnp.float32),
        mesh=mesh,
        scratch_types=[
            pltpu.VMEM((L * rows_w,), jnp.int32),
            pltpu.VMEM((4 * HIST_W,), jnp.float32),
            pltpu.SemaphoreType.DMA,
            pltpu.SemaphoreType.DMA,
            pltpu.SemaphoreType.DMA,
            pltpu.SemaphoreType.DMA,
        ],
        compiler_params=pltpu.CompilerParams(needs_layout_passes=False),
    )(xg)


def _tc_mlp_kernel(counts_ref, emb_ref, W1_ref, b1_ref, W2_ref, b2_ref,
                   W3_ref, b3_ref, out_ref):
    m = jnp.dot(counts_ref[...], emb_ref[...],
                preferred_element_type=jnp.float32) * (1.0 / L)
    h = jnp.maximum(jnp.dot(m, W1_ref[...],
                            preferred_element_type=jnp.float32) + b1_ref[...],
                    0.0)
    h2 = jnp.dot(h, W2_ref[...], preferred_element_type=jnp.float32) + b2_ref[...]
    h2 = jnp.where(h2 >= 0, h2, 0.01 * h2)
    out_ref[...] = jnp.dot(h2, W3_ref[...],
                           preferred_element_type=jnp.float32) + b3_ref[...]


def _tc_mlp(counts, emb_pad, W1, b1, W2, b2, W3, b3):
    nb = counts.shape[0]
    BM = 512
    grid = (nb // BM,)
    return pl.pallas_call(
        _tc_mlp_kernel,
        grid=grid,
        in_specs=[
            pl.BlockSpec((BM, VPAD), lambda i: (i, 0)),
            pl.BlockSpec((VPAD, DIM), lambda i: (0, 0)),
            pl.BlockSpec(W1.shape, lambda i: (0, 0)),
            pl.BlockSpec(b1.shape, lambda i: (0, 0)),
            pl.BlockSpec(W2.shape, lambda i: (0, 0)),
            pl.BlockSpec(b2.shape, lambda i: (0, 0)),
            pl.BlockSpec(W3.shape, lambda i: (0, 0)),
            pl.BlockSpec(b3.shape, lambda i: (0, 0)),
        ],
        out_specs=pl.BlockSpec((BM, 2), lambda i: (i, 0)),
        out_shape=jax.ShapeDtypeStruct((nb, 2), jnp.float32),
    )(counts, emb_pad, W1, b1, W2, b2, W3, b3)


def kernel(x, length, emb, W1, b1, W2, b2, W3, b3):
    del length
    emb_pad = jnp.pad(emb, ((0, VPAD - VOCAB), (0, 0)))
    b1r = b1.reshape(1, -1)
    b2r = b2.reshape(1, -1)
    b3r = b3.reshape(1, -1)

    rows_w = NB // NW
    outs = []
    for c in range(NCHUNK):
        xc = lax.slice_in_dim(x.astype(jnp.int32), c * NB, (c + 1) * NB, axis=0)
        xg = xc.reshape(NW, rows_w, L).transpose(0, 2, 1).reshape(NW, L * rows_w)
        counts = _sc_hist(xg).reshape(NB, VPAD)
        outs.append(_tc_mlp(counts, emb_pad, W1, b1r, W2, b2r, W3, b3r))
    return jnp.concatenate(outs, axis=0)


# speedup vs baseline: 1.2749x; 1.2066x over previous
import functools

import jax
import jax.numpy as jnp
from jax import lax
from jax.experimental import pallas as pl
from jax.experimental.pallas import tpu as pltpu
from jax.experimental.pallas import tpu_sc as plsc

B = 4096
L = 200
VOCAB = 1000
VPAD = 1024
DIM = 128
PACK = 65536.0

NC = 2
NS = 16
LANES = 16
NW = NC * NS
ROWS_W = B // NW
RPG = 2 * LANES
GROUPS = ROWS_W // RPG

HIST_W = LANES * VPAD


def _sc_hist_kernel(xg_hbm, counts_hbm, x_v, hist_v, sem0, sem1):
    wid = lax.axis_index("s") * NC + lax.axis_index("c")
    basep = wid * (ROWS_W // 2)

    pltpu.sync_copy(xg_hbm.at[wid], x_v)

    lane_off = lax.iota(jnp.int32, LANES) * VPAD
    ones = jnp.full((LANES,), 1.0, dtype=jnp.float32)
    hi_ones = jnp.full((LANES,), PACK, dtype=jnp.float32)
    zeros = jnp.zeros((LANES,), dtype=jnp.float32)
    sems = [sem0, sem1]
    pending = [None, None]

    for g in range(GROUPS):
        b = g % 2
        boff = b * HIST_W
        if pending[b] is not None:
            pending[b].wait()
        lane_off_g = lane_off + boff

        def _zero(i, _, boff=boff):
            for u in range(16):
                hist_v[pl.ds(boff + (i * 16 + u) * LANES, LANES)] = zeros
            return 0

        lax.fori_loop(0, HIST_W // LANES // 16, _zero, 0)

        def _scat(t, _, g=g, lane_off_g=lane_off_g):
            for u in range(8):
                pos = (t * 8 + u) * ROWS_W + g * RPG
                toks_lo = x_v[pl.ds(pos, LANES)]
                plsc.addupdate_scatter(hist_v, [lane_off_g + toks_lo], ones)
                toks_hi = x_v[pl.ds(pos + LANES, LANES)]
                plsc.addupdate_scatter(hist_v, [lane_off_g + toks_hi], hi_ones)
            return 0

        lax.fori_loop(0, L // 8, _scat, 0)

        pending[b] = pltpu.async_copy(
            hist_v.at[pl.ds(boff, HIST_W)],
            counts_hbm.at[pl.ds((basep + g * LANES) * VPAD, HIST_W)],
            sems[b])

    for p in pending:
        if p is not None:
            p.wait()


@jax.jit
def _sc_hist(xg):
    mesh = plsc.VectorSubcoreMesh(core_axis_name="c", subcore_axis_name="s")
    return pl.kernel(
        _sc_hist_kernel,
        out_type=jax.ShapeDtypeStruct(((B // 2) * VPAD,), jnp.float32),
        mesh=mesh,
        scratch_types=[
            pltpu.VMEM((L * ROWS_W,), jnp.int32),
            pltpu.VMEM((2 * HIST_W,), jnp.float32),
            pltpu.SemaphoreType.DMA,
            pltpu.SemaphoreType.DMA,
        ],
        compiler_params=pltpu.CompilerParams(needs_layout_passes=False),
    )(xg)


def _tc_mlp_kernel(counts_ref, emb_ref, W1_ref, b1_ref, W2_ref, b2_ref,
                   W3_ref, b3_ref, out_ref):
    c = counts_ref[...]
    hi = jnp.floor(c * (1.0 / PACK))
    lo = c - hi * PACK
    c2 = jnp.concatenate([lo, hi], axis=0)
    m = jnp.dot(c2, emb_ref[...], preferred_element_type=jnp.float32) * (1.0 / L)
    h = jnp.maximum(jnp.dot(m, W1_ref[...],
                            preferred_element_type=jnp.float32) + b1_ref[...],
                    0.0)
    h2 = jnp.dot(h, W2_ref[...], preferred_element_type=jnp.float32) + b2_ref[...]
    h2 = jnp.where(h2 >= 0, h2, 0.01 * h2)
    o = jnp.dot(h2, W3_ref[...], preferred_element_type=jnp.float32) + b3_ref[...]
    bm = c.shape[0]
    out_ref[...] = jnp.concatenate([o[:bm], o[bm:]], axis=1)


def _tc_mlp(counts, emb_pad, W1, b1, W2, b2, W3, b3):
    npk = counts.shape[0]
    BM = 512
    grid = (npk // BM,)
    return pl.pallas_call(
        _tc_mlp_kernel,
        grid=grid,
        in_specs=[
            pl.BlockSpec((BM, VPAD), lambda i: (i, 0)),
            pl.BlockSpec((VPAD, DIM), lambda i: (0, 0)),
            pl.BlockSpec(W1.shape, lambda i: (0, 0)),
            pl.BlockSpec(b1.shape, lambda i: (0, 0)),
            pl.BlockSpec(W2.shape, lambda i: (0, 0)),
            pl.BlockSpec(b2.shape, lambda i: (0, 0)),
            pl.BlockSpec(W3.shape, lambda i: (0, 0)),
            pl.BlockSpec(b3.shape, lambda i: (0, 0)),
        ],
        out_specs=pl.BlockSpec((BM, 4), lambda i: (i, 0)),
        out_shape=jax.ShapeDtypeStruct((npk, 4), jnp.float32),
    )(counts, emb_pad, W1, b1, W2, b2, W3, b3)


def kernel(x, length, emb, W1, b1, W2, b2, W3, b3):
    del length
    xg = x.astype(jnp.int32).reshape(NW, ROWS_W, L)
    xg = xg.transpose(0, 2, 1).reshape(NW, L * ROWS_W)
    counts = _sc_hist(xg).reshape(B // 2, VPAD)

    emb_pad = jnp.pad(emb, ((0, VPAD - VOCAB), (0, 0)))
    b1r = b1.reshape(1, -1)
    b2r = b2.reshape(1, -1)
    b3r = b3.reshape(1, -1)
    out4 = _tc_mlp(counts, emb_pad, W1, b1r, W2, b2r, W3, b3r)

    o = out4.reshape(B // RPG, LANES, 4)
    lo = o[:, :, 0:2]
    hi = o[:, :, 2:4]
    return jnp.stack([lo, hi], axis=1).reshape(B, 2)


# speedup vs baseline: 1.2767x; 1.0014x over previous
import functools

import jax
import jax.numpy as jnp
from jax import lax
from jax.experimental import pallas as pl
from jax.experimental.pallas import tpu as pltpu
from jax.experimental.pallas import tpu_sc as plsc

B = 4096
L = 200
VOCAB = 1000
VPAD = 1024
DIM = 128
PACK = 65536.0

NC = 2
NS = 16
LANES = 16
NW = NC * NS
ROWS_W = B // NW
RPG = 2 * LANES
GROUPS = ROWS_W // RPG

HIST_W = LANES * VPAD


def _sc_hist_kernel(xg_hbm, counts_hbm, x_v, hist_v, sem0, sem1):
    wid = lax.axis_index("s") * NC + lax.axis_index("c")
    basep = wid * (ROWS_W // 2)

    pltpu.sync_copy(xg_hbm.at[wid], x_v)

    lane_off = lax.iota(jnp.int32, LANES) * VPAD
    ones = jnp.full((LANES,), 1.0, dtype=jnp.float32)
    hi_ones = jnp.full((LANES,), PACK, dtype=jnp.float32)
    zeros = jnp.zeros((LANES,), dtype=jnp.float32)
    sems = [sem0, sem1]
    pending = [None, None]

    for g in range(GROUPS):
        b = g % 2
        boff = b * HIST_W
        if pending[b] is not None:
            pending[b].wait()
        lane_off_g = lane_off + boff

        def _zero(i, _, boff=boff):
            for u in range(16):
                hist_v[pl.ds(boff + (i * 16 + u) * LANES, LANES)] = zeros
            return 0

        lax.fori_loop(0, HIST_W // LANES // 16, _zero, 0)

        def _scat(t, _, g=g, lane_off_g=lane_off_g):
            for u in range(8):
                pos = (t * 8 + u) * ROWS_W + g * RPG
                toks_lo = x_v[pl.ds(pos, LANES)]
                plsc.addupdate_scatter(hist_v, [lane_off_g + toks_lo], ones)
                toks_hi = x_v[pl.ds(pos + LANES, LANES)]
                plsc.addupdate_scatter(hist_v, [lane_off_g + toks_hi], hi_ones)
            return 0

        lax.fori_loop(0, L // 8, _scat, 0)

        pending[b] = pltpu.async_copy(
            hist_v.at[pl.ds(boff, HIST_W)],
            counts_hbm.at[pl.ds((basep + g * LANES) * VPAD, HIST_W)],
            sems[b])

    for p in pending:
        if p is not None:
            p.wait()


@jax.jit
def _sc_hist(xg):
    mesh = plsc.VectorSubcoreMesh(core_axis_name="c", subcore_axis_name="s")
    return pl.kernel(
        _sc_hist_kernel,
        out_type=jax.ShapeDtypeStruct(((B // 2) * VPAD,), jnp.float32),
        mesh=mesh,
        scratch_types=[
            pltpu.VMEM((L * ROWS_W,), jnp.int32),
            pltpu.VMEM((2 * HIST_W,), jnp.float32),
            pltpu.SemaphoreType.DMA,
            pltpu.SemaphoreType.DMA,
        ],
        compiler_params=pltpu.CompilerParams(needs_layout_passes=False),
    )(xg)


def _tc_mlp_kernel(counts_ref, emb_ref, W1_ref, b1_ref, W2_ref,
                   b2_ref, W3_ref, b3_ref, out_ref):
    c = counts_ref[...]
    hi = jnp.floor(c * (1.0 / PACK))
    lo = c - hi * PACK
    c2 = jnp.concatenate([lo, hi], axis=0)
    m = jnp.dot(c2, emb_ref[...], preferred_element_type=jnp.float32)
    h = jnp.maximum(jnp.dot(m, W1_ref[...],
                            preferred_element_type=jnp.float32) + b1_ref[...],
                    0.0)
    h2 = jnp.dot(h, W2_ref[...], preferred_element_type=jnp.float32) + b2_ref[...]
    h2 = jnp.where(h2 >= 0, h2, 0.01 * h2)
    o = jnp.dot(h2, W3_ref[...], preferred_element_type=jnp.float32) + b3_ref[...]
    bm = c.shape[0]
    out_ref[...] = jnp.concatenate([o[:bm], o[bm:]], axis=1)


def _tc_mlp(counts, emb_s, W1, b1, W2, b2, W3, b3):
    npk = counts.shape[0]
    BM = 1024
    grid = (npk // BM,)
    return pl.pallas_call(
        _tc_mlp_kernel,
        grid=grid,
        in_specs=[
            pl.BlockSpec((BM, VPAD), lambda i: (i, 0)),
            pl.BlockSpec((VPAD, DIM), lambda i: (0, 0)),
            pl.BlockSpec(W1.shape, lambda i: (0, 0)),
            pl.BlockSpec(b1.shape, lambda i: (0, 0)),
            pl.BlockSpec(W2.shape, lambda i: (0, 0)),
            pl.BlockSpec(b2.shape, lambda i: (0, 0)),
            pl.BlockSpec(W3.shape, lambda i: (0, 0)),
            pl.BlockSpec(b3.shape, lambda i: (0, 0)),
        ],
        out_specs=pl.BlockSpec((BM, 4), lambda i: (i, 0)),
        out_shape=jax.ShapeDtypeStruct((npk, 4), jnp.float32),
    )(counts, emb_s, W1, b1, W2, b2, W3, b3)


def kernel(x, length, emb, W1, b1, W2, b2, W3, b3):
    del length
    xg = x.astype(jnp.int32).reshape(NW, ROWS_W, L)
    xg = xg.transpose(0, 2, 1).reshape(NW, L * ROWS_W)
    counts = _sc_hist(xg).reshape(B // 2, VPAD)

    emb_s = jnp.pad(emb, ((0, VPAD - VOCAB), (0, 0))) * (1.0 / L)
    b1r = b1.reshape(1, -1)
    b2r = b2.reshape(1, -1)
    b3r = b3.reshape(1, -1)
    out4 = _tc_mlp(counts, emb_s, W1, b1r, W2, b2r, W3, b3r)

    o = out4.reshape(B // RPG, LANES, 4)
    lo = o[:, :, 0:2]
    hi = o[:, :, 2:4]
    return jnp.stack([lo, hi], axis=1).reshape(B, 2)


# speedup vs baseline: 1.3047x; 1.0219x over previous
import functools

import jax
import jax.numpy as jnp
from jax import lax
from jax.experimental import pallas as pl
from jax.experimental.pallas import tpu as pltpu
from jax.experimental.pallas import tpu_sc as plsc

B = 4096
L = 200
VOCAB = 1000
VPAD = 1024
DIM = 128
PACK = 65536.0

NC = 2
NS = 16
LANES = 16
NW = NC * NS
ROWS_W = B // NW
RPG = 2 * LANES
GROUPS = ROWS_W // RPG

HIST_W = LANES * VPAD


def _sc_hist_kernel(xg_hbm, counts_hbm, x_v, hist_v, sem0, sem1, xsem):
    wid = lax.axis_index("s") * NC + lax.axis_index("c")
    basep = wid * (ROWS_W // 2)

    xcopy = pltpu.async_copy(xg_hbm.at[wid], x_v, xsem)

    lane_off = lax.iota(jnp.int32, LANES) * VPAD
    ones = jnp.full((LANES,), 1.0, dtype=jnp.float32)
    hi_ones = jnp.full((LANES,), PACK, dtype=jnp.float32)
    zeros = jnp.zeros((LANES,), dtype=jnp.float32)
    sems = [sem0, sem1]
    pending = [None, None]

    def _zero_buf(boff):
        def _zero(i, _, boff=boff):
            for u in range(16):
                hist_v[pl.ds(boff + (i * 16 + u) * LANES, LANES)] = zeros
            return 0

        lax.fori_loop(0, HIST_W // LANES // 16, _zero, 0)

    _zero_buf(0)
    _zero_buf(HIST_W)
    xcopy.wait()

    for g in range(GROUPS):
        b = g % 2
        boff = b * HIST_W
        if pending[b] is not None:
            pending[b].wait()
        lane_off_g = lane_off + boff

        if g >= 2:
            _zero_buf(boff)

        def _scat(t, _, g=g, lane_off_g=lane_off_g):
            for u in range(8):
                pos = (t * 8 + u) * ROWS_W + g * RPG
                toks_lo = x_v[pl.ds(pos, LANES)]
                plsc.addupdate_scatter(hist_v, [lane_off_g + toks_lo], ones)
                toks_hi = x_v[pl.ds(pos + LANES, LANES)]
                plsc.addupdate_scatter(hist_v, [lane_off_g + toks_hi], hi_ones)
            return 0

        lax.fori_loop(0, L // 8, _scat, 0)

        pending[b] = pltpu.async_copy(
            hist_v.at[pl.ds(boff, HIST_W)],
            counts_hbm.at[pl.ds((basep + g * LANES) * VPAD, HIST_W)],
            sems[b])

    for p in pending:
        if p is not None:
            p.wait()


@jax.jit
def _sc_hist(xg):
    mesh = plsc.VectorSubcoreMesh(core_axis_name="c", subcore_axis_name="s")
    return pl.kernel(
        _sc_hist_kernel,
        out_type=jax.ShapeDtypeStruct(((B // 2) * VPAD,), jnp.float32),
        mesh=mesh,
        scratch_types=[
            pltpu.VMEM((L * ROWS_W,), jnp.int32),
            pltpu.VMEM((2 * HIST_W,), jnp.float32),
            pltpu.SemaphoreType.DMA,
            pltpu.SemaphoreType.DMA,
            pltpu.SemaphoreType.DMA,
        ],
        compiler_params=pltpu.CompilerParams(needs_layout_passes=False),
    )(xg)


def _tc_mlp_kernel(counts_ref, emb_ref, W1_ref, b1_ref, W2_ref,
                   b2_ref, W3_ref, b3_ref, out_ref):
    c = counts_ref[...]
    hi = jnp.floor(c * (1.0 / PACK))
    lo = c - hi * PACK
    c2 = jnp.concatenate([lo, hi], axis=0)
    m = jnp.dot(c2, emb_ref[...], preferred_element_type=jnp.float32)
    h = jnp.maximum(jnp.dot(m, W1_ref[...],
                            preferred_element_type=jnp.float32) + b1_ref[...],
                    0.0)
    h2 = jnp.dot(h, W2_ref[...], preferred_element_type=jnp.float32) + b2_ref[...]
    h2 = jnp.where(h2 >= 0, h2, 0.01 * h2)
    o = jnp.dot(h2, W3_ref[...], preferred_element_type=jnp.float32) + b3_ref[...]
    bm = c.shape[0]
    out_ref[...] = jnp.concatenate([o[:bm], o[bm:]], axis=1)


def _tc_mlp(counts, emb_s, W1, b1, W2, b2, W3, b3):
    npk = counts.shape[0]
    BM = 1024
    grid = (npk // BM,)
    return pl.pallas_call(
        _tc_mlp_kernel,
        grid=grid,
        in_specs=[
            pl.BlockSpec((BM, VPAD), lambda i: (i, 0)),
            pl.BlockSpec((VPAD, DIM), lambda i: (0, 0)),
            pl.BlockSpec(W1.shape, lambda i: (0, 0)),
            pl.BlockSpec(b1.shape, lambda i: (0, 0)),
            pl.BlockSpec(W2.shape, lambda i: (0, 0)),
            pl.BlockSpec(b2.shape, lambda i: (0, 0)),
            pl.BlockSpec(W3.shape, lambda i: (0, 0)),
            pl.BlockSpec(b3.shape, lambda i: (0, 0)),
        ],
        out_specs=pl.BlockSpec((BM, 4), lambda i: (i, 0)),
        out_shape=jax.ShapeDtypeStruct((npk, 4), jnp.float32),
    )(counts, emb_s, W1, b1, W2, b2, W3, b3)


def kernel(x, length, emb, W1, b1, W2, b2, W3, b3):
    del length
    xg = x.astype(jnp.int32).reshape(NW, ROWS_W, L)
    xg = xg.transpose(0, 2, 1).reshape(NW, L * ROWS_W)
    counts = _sc_hist(xg).reshape(B // 2, VPAD)

    emb_s = jnp.pad(emb, ((0, VPAD - VOCAB), (0, 0))) * (1.0 / L)
    b1r = b1.reshape(1, -1)
    b2r = b2.reshape(1, -1)
    b3r = b3.reshape(1, -1)
    out4 = _tc_mlp(counts, emb_s, W1, b1r, W2, b2r, W3, b3r)

    o = out4.reshape(B // RPG, LANES, 4)
    lo = o[:, :, 0:2]
    hi = o[:, :, 2:4]
    return jnp.stack([lo, hi], axis=1).reshape(B, 2)
